# baseline (speedup 1.0000x reference)
import functools

import jax
import jax.numpy as jnp
from jax import lax
from jax.experimental import pallas as pl
from jax.experimental.pallas import tpu as pltpu
from jax.experimental.pallas import tpu_sc as plsc

SCALE = 30.0
NC = 2
NS = 16
L = 16
NW = NC * NS
TCB = 1024


def _fast_log(x):
    bits = lax.bitcast_convert_type(x, jnp.int32)
    e = (bits >> 23) - 127
    m = lax.bitcast_convert_type(
        (bits & 0x007FFFFF) | 0x3F800000, jnp.float32)
    big = m > 1.4142135623730951
    m = jnp.where(big, m * 0.5, m)
    e = jnp.where(big, e + 1, e)
    z = (m - 1.0) / (m + 1.0)
    z2 = z * z
    p = 2.0 * z * (1.0 + z2 * (1.0 / 3.0 + z2 * (0.2 + z2 * (1.0 / 7.0))))
    return e.astype(jnp.float32) * 0.6931471805599453 + p


def _pad_body(x_ref, out_ref, mx_ref):
    x = x_ref[...]
    out_ref[:, :x.shape[1]] = x
    mx_ref[...] = jnp.max(x, axis=1).reshape(TCB // 128, 128)


def _make_pad_kernel(B, C):
    return pl.pallas_call(
        _pad_body,
        grid=(B // TCB,),
        in_specs=[pl.BlockSpec((TCB, C), lambda i: (i, 0))],
        out_specs=[pl.BlockSpec((TCB, 128), lambda i: (i, 0)),
                   pl.BlockSpec((TCB // 128, 128), lambda i: (i, 0))],
        out_shape=[jax.ShapeDtypeStruct((B, 128), jnp.float32),
                   jax.ShapeDtypeStruct((B // 128, 128), jnp.float32)],
    )


def _make_sc_kernel(B, C):
    rows_w = B // NW
    groups = rows_w // L
    inv_b = 1.0 / B

    mesh = plsc.VectorSubcoreMesh(core_axis_name="c", subcore_axis_name="s")

    @functools.partial(
        pl.kernel,
        out_type=jax.ShapeDtypeStruct((NW, L), jnp.float32),
        mesh=mesh,
        compiler_params=pltpu.CompilerParams(needs_layout_passes=False),
        scratch_types=[
            pltpu.VMEM((rows_w, 128), jnp.float32),
            pltpu.VMEM((rows_w,), jnp.int32),
            pltpu.VMEM((rows_w // 128, 128), jnp.float32),
            pltpu.VMEM((C,), jnp.float32),
            pltpu.VMEM((L,), jnp.float32),
        ],
    )
    def body(padded_hbm, maxes_hbm, targets_hbm, mlist_hbm, out_hbm,
             logits_v, targets_v, mx_v, mlist_v, stage_v):
        cid = lax.axis_index("c")
        sid = lax.axis_index("s")
        wid = cid * NS + sid

        pltpu.sync_copy(padded_hbm.at[pl.ds(wid * rows_w, rows_w), :],
                        logits_v)
        pltpu.sync_copy(maxes_hbm.at[pl.ds(wid * (rows_w // 128),
                                           rows_w // 128), :], mx_v)
        pltpu.sync_copy(targets_hbm.at[pl.ds(wid * rows_w, rows_w)], targets_v)
        pltpu.sync_copy(mlist_hbm, mlist_v)

        lane = lax.iota(jnp.int32, 16)

        U = 10

        def group_body(g, acc):
            rows = g * L + lane
            t = plsc.load_gather(targets_v, [rows])
            mg = plsc.load_gather(mlist_v, [t])
            st = plsc.load_gather(logits_v, [rows, t])
            mx = plsc.load_gather(mx_v, [rows >> 7, rows & 127])

            def sum_body(jj, ss):
                j = jj * U
                out = []
                for u in range(U):
                    v = plsc.load_gather(logits_v, [rows, lane * 0 + (j + u)])
                    v = v - jnp.where(t == j + u, mg, 0.0)
                    out.append(ss[u] + jnp.exp((v - mx) * SCALE))
                return tuple(out)

            zero = jnp.zeros((L,), jnp.float32)
            ss = list(lax.fori_loop(0, C // U, sum_body, (zero,) * U))
            while len(ss) > 1:
                ss = [ss[i] + ss[i + 1] if i + 1 < len(ss)
                      else ss[i] for i in range(0, len(ss), 2)]
            s = ss[0]
            nll = SCALE * (mx - st + mg) + _fast_log(s)
            return acc + nll

        acc = lax.fori_loop(0, groups, group_body,
                            jnp.zeros((L,), jnp.float32))

        stage_v[...] = acc * inv_b
        pltpu.sync_copy(stage_v, out_hbm.at[wid])

    return body


def _finish_body(parts_ref, out_ref):
    out_ref[0, 0] = jnp.sum(parts_ref[...])


def kernel(logits, targets, m_list):
    B, C = logits.shape
    padded, maxes = _make_pad_kernel(B, C)(logits)
    sc = _make_sc_kernel(B, C)
    parts = sc(padded, maxes, targets, m_list)
    total = pl.pallas_call(
        _finish_body,
        out_shape=jax.ShapeDtypeStruct((1, 1), jnp.float32),
        out_specs=pl.BlockSpec(memory_space=pltpu.SMEM),
    )(parts)
    return total.reshape(())


# speedup vs baseline: 1.3725x; 1.3725x over previous
import functools

import jax
import jax.numpy as jnp
from jax import lax
from jax.experimental import pallas as pl
from jax.experimental.pallas import tpu as pltpu
from jax.experimental.pallas import tpu_sc as plsc

SCALE = 30.0
NC = 2
NS = 16
L = 16
NW = NC * NS
TCB = 1024


def _fast_log(x):
    bits = lax.bitcast_convert_type(x, jnp.int32)
    e = (bits >> 23) - 127
    m = lax.bitcast_convert_type(
        (bits & 0x007FFFFF) | 0x3F800000, jnp.float32)
    big = m > 1.4142135623730951
    m = jnp.where(big, m * 0.5, m)
    e = jnp.where(big, e + 1, e)
    z = (m - 1.0) / (m + 1.0)
    z2 = z * z
    p = 2.0 * z * (1.0 + z2 * (1.0 / 3.0 + z2 * (0.2 + z2 * (1.0 / 7.0))))
    return e.astype(jnp.float32) * 0.6931471805599453 + p


def _make_sc_kernel(B_sc, C, B):
    rows_w = B_sc // NW
    words_w = rows_w * C
    prows_w = words_w // 128
    groups = rows_w // L
    inv_b = 1.0 / B

    mesh = plsc.VectorSubcoreMesh(core_axis_name="c", subcore_axis_name="s")

    @functools.partial(
        pl.kernel,
        out_type=jax.ShapeDtypeStruct((NW, L), jnp.float32),
        mesh=mesh,
        compiler_params=pltpu.CompilerParams(needs_layout_passes=False),
        scratch_types=[
            pltpu.VMEM((prows_w, 128), jnp.float32),
            pltpu.VMEM((rows_w,), jnp.int32),
            pltpu.VMEM((C,), jnp.float32),
            pltpu.VMEM((L,), jnp.float32),
        ],
    )
    def body(logits_hbm, targets_hbm, mlist_hbm, out_hbm,
             logits_v, targets_v, mlist_v, stage_v):
        cid = lax.axis_index("c")
        sid = lax.axis_index("s")
        wid = cid * NS + sid

        pltpu.sync_copy(logits_hbm.at[pl.ds(wid * prows_w, prows_w), :],
                        logits_v)
        pltpu.sync_copy(targets_hbm.at[pl.ds(wid * rows_w, rows_w)], targets_v)
        pltpu.sync_copy(mlist_hbm, mlist_v)

        lane = lax.iota(jnp.int32, 16)

        U = 10

        def gather_flat(o):
            return plsc.load_gather(logits_v, [o >> 7, o & 127])

        def group_body(g, acc):
            rows = g * L + lane
            ob = rows * C
            t = plsc.load_gather(targets_v, [rows])
            mg = plsc.load_gather(mlist_v, [t])
            st = gather_flat(ob + t)

            def max_body(jj, ms):
                j = jj * U
                return tuple(
                    jnp.maximum(ms[u], gather_flat(ob + (j + u)))
                    for u in range(U))

            neg = jnp.full((L,), -jnp.inf, jnp.float32)
            ms = list(lax.fori_loop(0, C // U, max_body, (neg,) * U))
            while len(ms) > 1:
                ms = [jnp.maximum(ms[i], ms[i + 1]) if i + 1 < len(ms)
                      else ms[i] for i in range(0, len(ms), 2)]
            mx = ms[0]

            def sum_body(jj, ss):
                j = jj * U
                out = []
                for u in range(U):
                    v = gather_flat(ob + (j + u))
                    v = v - jnp.where(t == j + u, mg, 0.0)
                    out.append(ss[u] + jnp.exp((v - mx) * SCALE))
                return tuple(out)

            zero = jnp.zeros((L,), jnp.float32)
            ss = list(lax.fori_loop(0, C // U, sum_body, (zero,) * U))
            while len(ss) > 1:
                ss = [ss[i] + ss[i + 1] if i + 1 < len(ss)
                      else ss[i] for i in range(0, len(ss), 2)]
            s = ss[0]
            nll = SCALE * (mx - st + mg) + _fast_log(s)
            return acc + nll

        acc = lax.fori_loop(0, groups, group_body,
                            jnp.zeros((L,), jnp.float32))

        stage_v[...] = acc * inv_b
        pltpu.sync_copy(stage_v, out_hbm.at[wid])

    return body


def _make_tc_half_kernel(B, C, row0):
    inv_b = 1.0 / B
    nsteps = (B - row0) // TCB
    blk0 = row0 // TCB

    def tc_body(x_ref, t_ref, m_ref, out_ref):
        i = pl.program_id(0)
        x = x_ref[...] * SCALE
        t = t_ref[0, 0, :]
        ml = m_ref[0, 0, :]
        iota_c = lax.broadcasted_iota(jnp.int32, (TCB, C), 1)
        mask = iota_c == t[:, None]
        mg = jnp.sum(jnp.where(mask, ml[None, :] * SCALE, 0.0), axis=1)
        z = jnp.where(mask, x - mg[:, None], x)
        zt = jnp.sum(jnp.where(mask, z, 0.0), axis=1)
        mx = jnp.max(z, axis=1)
        s = jnp.sum(jnp.exp(z - mx[:, None]), axis=1)
        nll = mx + jnp.log(s) - zt

        @pl.when(i == 0)
        def _():
            out_ref[0, 0] = 0.0

        out_ref[0, 0] += jnp.sum(nll) * inv_b

    return pl.pallas_call(
        tc_body,
        grid=(nsteps,),
        in_specs=[
            pl.BlockSpec((TCB, C), lambda i: (i + blk0, 0)),
            pl.BlockSpec((1, 1, TCB), lambda i: (i + blk0, 0, 0)),
            pl.BlockSpec((1, 1, C), lambda i: (0, 0, 0)),
        ],
        out_specs=pl.BlockSpec(memory_space=pltpu.SMEM),
        out_shape=jax.ShapeDtypeStruct((1, 1), jnp.float32),
    )


def _finish_body(parts_ref, tc_ref, out_ref):
    out_ref[0, 0] = jnp.sum(parts_ref[...]) + tc_ref[0, 0]


def kernel(logits, targets, m_list):
    B, C = logits.shape
    B_sc = B // 2
    sc = _make_sc_kernel(B_sc, C, B)
    parts = sc(logits[:B_sc].reshape(B_sc * C // 128, 128),
               targets[:B_sc], m_list)
    tc_half = _make_tc_half_kernel(B, C, B_sc)
    tc_part = tc_half(logits, targets.reshape(B // TCB, 1, TCB),
                      m_list.reshape(1, 1, C))
    total = pl.pallas_call(
        _finish_body,
        out_shape=jax.ShapeDtypeStruct((1, 1), jnp.float32),
        out_specs=pl.BlockSpec(memory_space=pltpu.SMEM),
    )(parts, tc_part)
    return total.reshape(())
